# baseline (speedup 1.0000x reference)
import functools

import jax
import jax.numpy as jnp
from jax import lax
from jax.experimental import pallas as pl
from jax.experimental.pallas import tpu as pltpu
from jax.experimental.pallas import tpu_sc as plsc

_TIME = 288
_F = 64
_T = 12
_CH = 128
_NB = 512


def _sc_gather(dayf, wkf, time_day, time_week):
    B, N = dayf.shape
    mesh = plsc.VectorSubcoreMesh(core_axis_name="c", subcore_axis_name="s")

    @functools.partial(
        pl.kernel,
        mesh=mesh,
        out_type=jax.ShapeDtypeStruct((B * N, 128), jnp.float32),
        scratch_types=[
            pltpu.VMEM((N,), jnp.float32),
            pltpu.VMEM((N,), jnp.float32),
            pltpu.VMEM((N,), jnp.int32),
            pltpu.VMEM((N,), jnp.int32),
            pltpu.VMEM((_CH, 128), jnp.float32),
            pltpu.VMEM((_CH, 128), jnp.float32),
            pltpu.SemaphoreType.DMA,
            pltpu.SemaphoreType.DMA,
        ],
    )
    def k(dayf_hbm, wkf_hbm, td_hbm, tw_hbm, out_hbm,
          xd_v, xw_v, di_v, wi_v, dbuf, wbuf, sem1, sem2):
        wid = lax.axis_index("s") * 2 + lax.axis_index("c")
        pltpu.sync_copy(dayf_hbm.at[wid], xd_v)
        pltpu.sync_copy(wkf_hbm.at[wid], xw_v)

        def idxbody(j, carry):
            v = xd_v[pl.ds(j * 16, 16)]
            di_v[pl.ds(j * 16, 16)] = jnp.clip((v * float(_TIME)).astype(jnp.int32), 0, _TIME - 1)
            w = xw_v[pl.ds(j * 16, 16)]
            wi_v[pl.ds(j * 16, 16)] = jnp.clip(w.astype(jnp.int32), 0, 6)
            return carry

        lax.fori_loop(0, N // 16, idxbody, 0)

        def chbody(c, carry):
            cpd = pltpu.async_copy(td_hbm.at[di_v.at[pl.ds(c * _CH, _CH)]], dbuf, sem1)
            cpw = pltpu.async_copy(tw_hbm.at[wi_v.at[pl.ds(c * _CH, _CH)]], wbuf, sem2)
            cpd.wait()
            cpw.wait()

            def addbody(r, carry2):
                for q in range(_F // 16):
                    sl = pl.ds(q * 16, 16)
                    dbuf[r, sl] = dbuf[r, sl] + wbuf[r, sl]
                return carry2

            lax.fori_loop(0, _CH, addbody, 0)
            pltpu.sync_copy(dbuf, out_hbm.at[pl.ds(wid * N + c * _CH, _CH)])
            return carry

        lax.fori_loop(0, N // _CH, chbody, 0)

    return k(dayf, wkf, time_day, time_week)


def _tc_body(emb_ref, out_ref):
    e = emb_ref[0, :, : _F]
    eye = (
        jax.lax.broadcasted_iota(jnp.int32, (_F, _F), 0)
        == jax.lax.broadcasted_iota(jnp.int32, (_F, _F), 1)
    ).astype(jnp.float32)
    et = jax.lax.dot_general(
        eye, e, (((1,), (1,)), ((), ())), preferred_element_type=jnp.float32
    )
    out_ref[0] = jnp.broadcast_to(et[None], (_T, _F, _NB))


def kernel(x, time_day, time_week):
    B, C, N, T = x.shape
    F = time_day.shape[1]
    dayf = x[:, 1, :, T - 1]
    wkf = x[:, 2, :, T - 1]
    tdp = jnp.pad(time_day, ((0, 0), (0, 128 - F)))
    twp = jnp.pad(time_week, ((0, 1), (0, 128 - F)))

    emb = _sc_gather(dayf, wkf, tdp, twp)
    emb3 = emb.reshape(B, N, 128)

    grid = (B, N // _NB)
    out_tfn = pl.pallas_call(
        _tc_body,
        grid=grid,
        in_specs=[pl.BlockSpec((1, _NB, 128), lambda b, n: (b, n, 0))],
        out_specs=pl.BlockSpec((1, T, F, _NB), lambda b, n: (b, 0, 0, n)),
        out_shape=jax.ShapeDtypeStruct((B, T, F, N), jnp.float32),
    )(emb3)
    return out_tfn.transpose(0, 2, 3, 1)


# speedup vs baseline: 12.1778x; 12.1778x over previous
import functools

import jax
import jax.numpy as jnp
from jax import lax
from jax.experimental import pallas as pl
from jax.experimental.pallas import tpu as pltpu
from jax.experimental.pallas import tpu_sc as plsc

_TIME = 288
_F = 64
_T = 12
_CH = 128
_NB = 2048


def _sc_gather(dayf, wkf, time_day, time_week):
    B, N = dayf.shape
    mesh = plsc.VectorSubcoreMesh(core_axis_name="c", subcore_axis_name="s")

    @functools.partial(
        pl.kernel,
        mesh=mesh,
        compiler_params=pltpu.CompilerParams(needs_layout_passes=False),
        out_type=jax.ShapeDtypeStruct((_F, B * N), jnp.float32),
        scratch_types=[
            pltpu.VMEM((_TIME * 128,), jnp.float32),
            pltpu.VMEM((8 * 128,), jnp.float32),
            pltpu.VMEM((N,), jnp.float32),
            pltpu.VMEM((N,), jnp.float32),
            pltpu.VMEM((N,), jnp.int32),
            pltpu.VMEM((N,), jnp.int32),
            pltpu.VMEM((_F, _CH), jnp.float32),
        ],
    )
    def k(dayf_hbm, wkf_hbm, td_hbm, tw_hbm, out_hbm,
          td_v, tw_v, xd_v, xw_v, di_v, wi_v, obuf):
        wid = lax.axis_index("s") * 2 + lax.axis_index("c")
        pltpu.sync_copy(td_hbm, td_v)
        pltpu.sync_copy(tw_hbm, tw_v)
        pltpu.sync_copy(dayf_hbm.at[wid], xd_v)
        pltpu.sync_copy(wkf_hbm.at[wid], xw_v)

        def idxbody(j, carry):
            sl = pl.ds(j * 16, 16)
            v = xd_v[sl]
            di_v[sl] = jnp.clip((v * float(_TIME)).astype(jnp.int32), 0, _TIME - 1) * 128
            w = xw_v[sl]
            wi_v[sl] = jnp.clip(w.astype(jnp.int32), 0, 6) * 128
            return carry

        lax.fori_loop(0, N // 16, idxbody, 0)

        def chbody(c, carry):
            for g in range(_CH // 16):
                dbase = di_v[pl.ds(c * _CH + g * 16, 16)]
                wbase = wi_v[pl.ds(c * _CH + g * 16, 16)]
                for f in range(_F):
                    dval = plsc.load_gather(td_v, [dbase + f])
                    wval = plsc.load_gather(tw_v, [wbase + f])
                    obuf[f, pl.ds(g * 16, 16)] = dval + wval
            pltpu.sync_copy(obuf, out_hbm.at[:, pl.ds(wid * N + c * _CH, _CH)])
            return carry

        lax.fori_loop(0, N // _CH, chbody, 0)

    return k(dayf, wkf, time_day.reshape(-1), time_week.reshape(-1))


def _tc_body(emb_ref, out_ref):
    out_ref[0] = jnp.broadcast_to(emb_ref[...][None], (_T, _F, _NB))


def kernel(x, time_day, time_week):
    B, C, N, T = x.shape
    F = time_day.shape[1]
    dayf = x[:, 1, :, T - 1]
    wkf = x[:, 2, :, T - 1]
    tdp = jnp.pad(time_day, ((0, 0), (0, 128 - F)))
    twp = jnp.pad(time_week, ((0, 1), (0, 128 - F)))

    emb_t = _sc_gather(dayf, wkf, tdp, twp)

    grid = (B, N // _NB)
    out_tfn = pl.pallas_call(
        _tc_body,
        grid=grid,
        in_specs=[
            pl.BlockSpec((F, _NB), lambda b, n: (0, b * (2048 // _NB) + n)),
        ],
        out_specs=pl.BlockSpec((1, T, F, _NB), lambda b, n: (b, 0, 0, n)),
        out_shape=jax.ShapeDtypeStruct((B, T, F, N), jnp.float32),
    )(emb_t)
    return out_tfn.transpose(0, 2, 3, 1)
